# Initial kernel scaffold; baseline (speedup 1.0000x reference)
import jax
import jax.numpy as jnp
from jax.experimental import pallas as pl


def kernel(x, edge_index, edge_attr, Wl1, bl1, Wr1, Wl2, bl2, Wr2):
    raise NotImplementedError("write your pallas kernel here")


# speedup vs baseline: 3.9927x; 3.9927x over previous
import functools

import jax
import jax.numpy as jnp
from jax import lax
from jax.experimental import pallas as pl
from jax.experimental.pallas import tpu as pltpu
from jax.experimental.pallas import tpu_sc as plsc

N = 10000
E = 320000
D = 128

NC = 2
NS = 16
NW = NC * NS
EPT = E // NW
C = 80
NCHUNK = EPT // C
NPAD = 10240
ROWS_PER_SUB = NPAD // NS


def _fill(ref, rows, value):
    @pl.loop(0, rows)
    def _(r):
        for g in range(D // 16):
            ref[r, pl.ds(g * 16, 16)] = jnp.full((16,), value, jnp.float32)


def _sc_edge_body(z_hbm, src_hbm, dst_hbm, attr_hbm, sum_out,
                  src_c, dst_c, zrow, attr_b, isem, jsem, gsem, asem, acc):
    c = lax.axis_index("c")
    s = lax.axis_index("s")
    wid = c * NS + s

    _fill(zrow, C, 0.0)

    row0 = s * ROWS_PER_SUB
    for k in range(ROWS_PER_SUB // C):
        pltpu.sync_copy(zrow, acc.at[pl.ds(row0 + k * C, C)])

    plsc.subcore_barrier()

    ebase = wid * EPT

    @pl.loop(0, NCHUNK)
    def _(j):
        si = pltpu.async_copy(src_hbm.at[wid, j], src_c, isem)
        di = pltpu.async_copy(dst_hbm.at[wid, j], dst_c, jsem)
        si.wait()
        g = pltpu.async_copy(z_hbm.at[src_c], zrow, gsem)
        a = pltpu.async_copy(attr_hbm.at[pl.ds(ebase + j * C, C)], attr_b, asem)
        g.wait()
        a.wait()

        @pl.loop(0, C)
        def _(r):
            for gc in range(D // 16):
                sl = pl.ds(gc * 16, 16)
                zrow[r, sl] = jnp.maximum(zrow[r, sl] + attr_b[r, sl], 0.0)

        di.wait()
        pltpu.sync_copy(zrow, acc.at[dst_c], add=True)

    plsc.subcore_barrier()

    out_sl = pl.ds(row0, ROWS_PER_SUB)
    pltpu.sync_copy(acc.at[out_sl], sum_out.at[c, out_sl])


_sc_layer = pl.kernel(
    _sc_edge_body,
    out_type=[jax.ShapeDtypeStruct((NC, NPAD, D), jnp.float32)],
    mesh=plsc.VectorSubcoreMesh(core_axis_name="c", subcore_axis_name="s"),
    scratch_types=[
        pltpu.VMEM((C,), jnp.int32),
        pltpu.VMEM((C,), jnp.int32),
        pltpu.VMEM((C, D), jnp.float32),
        pltpu.VMEM((C, D), jnp.float32),
        pltpu.SemaphoreType.DMA,
        pltpu.SemaphoreType.DMA,
        pltpu.SemaphoreType.DMA,
        pltpu.SemaphoreType.DMA,
        pltpu.VMEM_SHARED((NPAD, D), jnp.float32),
    ],
)


def _sc_degree_body(dst_hbm, deg_out, dst_c, ones_b, jsem, acc):
    c = lax.axis_index("c")
    s = lax.axis_index("s")
    wid = c * NS + s

    _fill(ones_b, C, 0.0)
    row0 = s * ROWS_PER_SUB
    for k in range(ROWS_PER_SUB // C):
        pltpu.sync_copy(ones_b, acc.at[pl.ds(row0 + k * C, C)])
    _fill(ones_b, C, 1.0)

    plsc.subcore_barrier()

    @pl.loop(0, NCHUNK)
    def _(j):
        pltpu.async_copy(dst_hbm.at[wid, j], dst_c, jsem).wait()
        pltpu.sync_copy(ones_b, acc.at[dst_c], add=True)

    plsc.subcore_barrier()

    out_sl = pl.ds(row0, ROWS_PER_SUB)
    pltpu.sync_copy(acc.at[out_sl], deg_out.at[c, out_sl])


_sc_degree = pl.kernel(
    _sc_degree_body,
    out_type=[jax.ShapeDtypeStruct((NC, NPAD, D), jnp.float32)],
    mesh=plsc.VectorSubcoreMesh(core_axis_name="c", subcore_axis_name="s"),
    scratch_types=[
        pltpu.VMEM((C,), jnp.int32),
        pltpu.VMEM((C, D), jnp.float32),
        pltpu.SemaphoreType.DMA,
        pltpu.VMEM_SHARED((NPAD, D), jnp.float32),
    ],
)

_TCR = 2000


def _tc_body(relu_out, p_ref, deg_ref, z_ref, wl_ref, bl_ref, wr_ref, o_ref):
    ssum = p_ref[0] + p_ref[1]
    cnt = deg_ref[0, :, 0] + deg_ref[1, :, 0]
    recip = 1.0 / jnp.maximum(cnt, 1.0)
    agg = ssum * recip[:, None]
    dn = (((1,), (1,)), ((), ()))
    y = (lax.dot_general(agg, wl_ref[...], dn,
                         preferred_element_type=jnp.float32)
         + bl_ref[...]
         + lax.dot_general(z_ref[...], wr_ref[...], dn,
                           preferred_element_type=jnp.float32))
    o_ref[...] = jnp.maximum(y, 0.0) if relu_out else y


def _tc_combine(psum, deg, z, Wl, bl, Wr, relu_out):
    return pl.pallas_call(
        functools.partial(_tc_body, relu_out),
        grid=(N // _TCR,),
        in_specs=[
            pl.BlockSpec((NC, _TCR, D), lambda i: (0, i, 0)),
            pl.BlockSpec((NC, _TCR, D), lambda i: (0, i, 0)),
            pl.BlockSpec((_TCR, D), lambda i: (i, 0)),
            pl.BlockSpec((D, D), lambda i: (0, 0)),
            pl.BlockSpec((1, D), lambda i: (0, 0)),
            pl.BlockSpec((D, D), lambda i: (0, 0)),
        ],
        out_specs=pl.BlockSpec((_TCR, D), lambda i: (i, 0)),
        out_shape=jax.ShapeDtypeStruct((N, D), jnp.float32),
    )(psum, deg, z, Wl, bl.reshape(1, D), Wr)


def kernel(x, edge_index, edge_attr, Wl1, bl1, Wr1, Wl2, bl2, Wr2):
    src3 = edge_index[0].reshape(NW, NCHUNK, C)
    dst3 = edge_index[1].reshape(NW, NCHUNK, C)
    (deg,) = _sc_degree(dst3)
    (psum1,) = _sc_layer(x, src3, dst3, edge_attr)
    z1 = _tc_combine(psum1, deg, x, Wl1, bl1, Wr1, True)
    (psum2,) = _sc_layer(z1, src3, dst3, edge_attr)
    return _tc_combine(psum2, deg, z1, Wl2, bl2, Wr2, False)


# speedup vs baseline: 7.4985x; 1.8780x over previous
import functools

import jax
import jax.numpy as jnp
from jax import lax
from jax.experimental import pallas as pl
from jax.experimental.pallas import tpu as pltpu
from jax.experimental.pallas import tpu_sc as plsc

N = 10000
E = 320000
D = 128

NC = 2
NS = 16
NW = NC * NS
EPT = E // NW
C = 80
NCHUNK = EPT // C
NPAD = 10240
ROWS_PER_SUB = NPAD // NS


def _fill(ref, rows, value):
    @pl.loop(0, rows)
    def _(r):
        for g in range(D // 16):
            ref[r, pl.ds(g * 16, 16)] = jnp.full((16,), value, jnp.float32)


def _sc_edge_body(z_hbm, src_hbm, dst_hbm, attr_hbm, sum_out,
                  src_c, dst_c, zrow, attr_b,
                  isem, jsem, gsem, asem, ssem, acc):
    c = lax.axis_index("c")
    s = lax.axis_index("s")
    wid = c * NS + s

    _fill(zrow[0], C, 0.0)

    row0 = s * ROWS_PER_SUB
    for k in range(ROWS_PER_SUB // C):
        pltpu.sync_copy(zrow[0], acc.at[pl.ds(row0 + k * C, C)])

    plsc.subcore_barrier()

    ebase = wid * EPT

    def idx_slice(arr, j):
        return arr.at[pl.ds(ebase + j * C, C)]

    def idx_copy(arr, j, buf, sem):
        return pltpu.async_copy(idx_slice(arr, j), buf, sem)

    def scatter_wait(b):
        pltpu.make_async_copy(zrow[b], acc.at[dst_c[b]], ssem[b]).wait()

    def emit_step(j, b, *, wait_prev=True, start_next=True,
                  prefetch_src2=True):
        nb = 1 - b
        if wait_prev:
            scatter_wait(nb)
        if start_next:
            idx_copy(dst_hbm, j + 1, dst_c[nb], jsem[nb])
            pltpu.make_async_copy(
                idx_slice(src_hbm, j + 1), src_c[nb], isem[nb]).wait()
            pltpu.async_copy(z_hbm.at[src_c[nb]], zrow[nb], gsem[nb])
            pltpu.async_copy(attr_hbm.at[pl.ds(ebase + (j + 1) * C, C)],
                             attr_b[nb], asem[nb])
        pltpu.make_async_copy(z_hbm.at[src_c[b]], zrow[b], gsem[b]).wait()
        pltpu.make_async_copy(attr_hbm.at[pl.ds(ebase + j * C, C)],
                              attr_b[b], asem[b]).wait()

        @pl.loop(0, C)
        def _(r):
            for gc in range(D // 16):
                sl = pl.ds(gc * 16, 16)
                zrow[b][r, sl] = jnp.maximum(
                    zrow[b][r, sl] + attr_b[b][r, sl], 0.0)

        if prefetch_src2:
            idx_copy(src_hbm, j + 2, src_c[b], isem[b])
        pltpu.make_async_copy(idx_slice(dst_hbm, j), dst_c[b], jsem[b]).wait()
        pltpu.async_copy(zrow[b], acc.at[dst_c[b]], ssem[b], add=True)

    idx_copy(src_hbm, 0, src_c[0], isem[0])
    idx_copy(src_hbm, 1, src_c[1], isem[1])
    idx_copy(dst_hbm, 0, dst_c[0], jsem[0])
    pltpu.make_async_copy(idx_slice(src_hbm, 0), src_c[0], isem[0]).wait()
    pltpu.async_copy(z_hbm.at[src_c[0]], zrow[0], gsem[0])
    pltpu.async_copy(attr_hbm.at[pl.ds(ebase, C)], attr_b[0], asem[0])

    emit_step(0, 0, wait_prev=False)

    npairs = (NCHUNK - 5) // 2

    @pl.loop(0, npairs)
    def _(g):
        emit_step(1 + 2 * g, 1)
        emit_step(2 + 2 * g, 0)

    for j in range(1 + 2 * npairs, NCHUNK):
        emit_step(j, j % 2,
                  start_next=j + 1 < NCHUNK,
                  prefetch_src2=j + 2 < NCHUNK)

    scatter_wait((NCHUNK - 1) % 2)

    plsc.subcore_barrier()

    out_sl = pl.ds(row0, ROWS_PER_SUB)
    pltpu.sync_copy(acc.at[out_sl], sum_out.at[c, out_sl])


_sc_layer = pl.kernel(
    _sc_edge_body,
    out_type=[jax.ShapeDtypeStruct((NC, NPAD, D), jnp.float32)],
    mesh=plsc.VectorSubcoreMesh(core_axis_name="c", subcore_axis_name="s"),
    scratch_types=[
        [pltpu.VMEM((C,), jnp.int32)] * 2,
        [pltpu.VMEM((C,), jnp.int32)] * 2,
        [pltpu.VMEM((C, D), jnp.float32)] * 2,
        [pltpu.VMEM((C, D), jnp.float32)] * 2,
        [pltpu.SemaphoreType.DMA] * 2,
        [pltpu.SemaphoreType.DMA] * 2,
        [pltpu.SemaphoreType.DMA] * 2,
        [pltpu.SemaphoreType.DMA] * 2,
        [pltpu.SemaphoreType.DMA] * 2,
        pltpu.VMEM_SHARED((NPAD, D), jnp.float32),
    ],
)


def _sc_degree_body(dst_hbm, deg_out, dst_c, ones_b, jsem, ssem, acc):
    c = lax.axis_index("c")
    s = lax.axis_index("s")
    wid = c * NS + s

    _fill(ones_b, C, 0.0)
    row0 = s * ROWS_PER_SUB
    for k in range(ROWS_PER_SUB // C):
        pltpu.sync_copy(ones_b, acc.at[pl.ds(row0 + k * C, C)])
    _fill(ones_b, C, 1.0)

    plsc.subcore_barrier()

    ebase = wid * EPT

    def idx_slice(j):
        return dst_hbm.at[pl.ds(ebase + j * C, C)]

    def step(j, b, *, wait_prev=True, issue_next=True):
        nb = 1 - b
        if wait_prev:
            pltpu.make_async_copy(ones_b, acc.at[dst_c[nb]], ssem[nb]).wait()
        if issue_next:
            pltpu.async_copy(idx_slice(j + 1), dst_c[nb], jsem[nb])
        pltpu.make_async_copy(idx_slice(j), dst_c[b], jsem[b]).wait()
        pltpu.async_copy(ones_b, acc.at[dst_c[b]], ssem[b], add=True)

    pltpu.async_copy(idx_slice(0), dst_c[0], jsem[0])
    step(0, 0, wait_prev=False)

    npairs = (NCHUNK - 3) // 2

    @pl.loop(0, npairs)
    def _(g):
        step(2 * g + 1, 1)
        step(2 * g + 2, 0)

    for j in range(1 + 2 * npairs, NCHUNK):
        step(j, j % 2, issue_next=j + 1 < NCHUNK)

    pltpu.make_async_copy(
        ones_b, acc.at[dst_c[(NCHUNK - 1) % 2]], ssem[(NCHUNK - 1) % 2]).wait()

    plsc.subcore_barrier()

    out_sl = pl.ds(row0, ROWS_PER_SUB)
    pltpu.sync_copy(acc.at[out_sl], deg_out.at[c, out_sl])


_sc_degree = pl.kernel(
    _sc_degree_body,
    out_type=[jax.ShapeDtypeStruct((NC, NPAD, D), jnp.float32)],
    mesh=plsc.VectorSubcoreMesh(core_axis_name="c", subcore_axis_name="s"),
    scratch_types=[
        [pltpu.VMEM((C,), jnp.int32)] * 2,
        pltpu.VMEM((C, D), jnp.float32),
        [pltpu.SemaphoreType.DMA] * 2,
        [pltpu.SemaphoreType.DMA] * 2,
        pltpu.VMEM_SHARED((NPAD, D), jnp.float32),
    ],
)

_TCR = 2000


def _tc_body(relu_out, p_ref, deg_ref, z_ref, wl_ref, bl_ref, wr_ref, o_ref):
    ssum = p_ref[0] + p_ref[1]
    cnt = deg_ref[0, :, 0] + deg_ref[1, :, 0]
    recip = 1.0 / jnp.maximum(cnt, 1.0)
    agg = ssum * recip[:, None]
    dn = (((1,), (1,)), ((), ()))
    y = (lax.dot_general(agg, wl_ref[...], dn,
                         preferred_element_type=jnp.float32)
         + bl_ref[...]
         + lax.dot_general(z_ref[...], wr_ref[...], dn,
                           preferred_element_type=jnp.float32))
    o_ref[...] = jnp.maximum(y, 0.0) if relu_out else y


def _tc_combine(psum, deg, z, Wl, bl, Wr, relu_out):
    return pl.pallas_call(
        functools.partial(_tc_body, relu_out),
        grid=(N // _TCR,),
        in_specs=[
            pl.BlockSpec((NC, _TCR, D), lambda i: (0, i, 0)),
            pl.BlockSpec((NC, _TCR, D), lambda i: (0, i, 0)),
            pl.BlockSpec((_TCR, D), lambda i: (i, 0)),
            pl.BlockSpec((D, D), lambda i: (0, 0)),
            pl.BlockSpec((1, D), lambda i: (0, 0)),
            pl.BlockSpec((D, D), lambda i: (0, 0)),
        ],
        out_specs=pl.BlockSpec((_TCR, D), lambda i: (i, 0)),
        out_shape=jax.ShapeDtypeStruct((N, D), jnp.float32),
    )(psum, deg, z, Wl, bl.reshape(1, D), Wr)


def kernel(x, edge_index, edge_attr, Wl1, bl1, Wr1, Wl2, bl2, Wr2):
    src1 = edge_index[0]
    dst1 = edge_index[1]
    (deg,) = _sc_degree(dst1)
    (psum1,) = _sc_layer(x, src1, dst1, edge_attr)
    z1 = _tc_combine(psum1, deg, x, Wl1, bl1, Wr1, True)
    (psum2,) = _sc_layer(z1, src1, dst1, edge_attr)
    return _tc_combine(psum2, deg, z1, Wl2, bl2, Wr2, False)
